# baseline (speedup 1.0000x reference)
import functools

import jax
import jax.numpy as jnp
from jax import lax
from jax.experimental import pallas as pl
from jax.experimental.pallas import tpu as pltpu
from jax.experimental.pallas import tpu_sc as plsc

_KTOP = 2
_NEG = float(jnp.finfo(jnp.float32).min)
_BB = 4096


def _stage1(x_ref, wg_ref, bg_ref, w1_ref, b1_ref, hs_ref, lt_ref):
    x = x_ref[...]
    logits = jnp.dot(x, wg_ref[...], preferred_element_type=jnp.float32)
    logits = logits + bg_ref[...]
    lt_ref[...] = logits.T
    xb = x.astype(jnp.bfloat16)
    h = jnp.dot(xb, w1_ref[...].astype(jnp.bfloat16),
                preferred_element_type=jnp.float32)
    h = h + b1_ref[...]
    hs_ref[...] = (h * (1.0 + lax.erf(h * 0.7071067811865476))
                   ).astype(jnp.bfloat16)


def _sc_gating(lt_hbm, gt_hbm, lbuf, gbuf):
    E = lt_hbm.shape[0]
    B = lt_hbm.shape[1]
    nc = 2
    bw = B // 32
    wid = lax.axis_index("s") * nc + lax.axis_index("c")
    base = wid * bw
    pltpu.sync_copy(lt_hbm.at[:, pl.ds(base, bw)], lbuf)
    for t in range(bw // 16):
        v = [lbuf[e, pl.ds(t * 16, 16)] for e in range(E)]
        m1 = v[0]
        for e in range(1, E):
            m1 = jnp.maximum(m1, v[e])
        i1 = jnp.zeros((16,), jnp.float32)
        for e in range(E - 1, -1, -1):
            i1 = jnp.where(v[e] == m1, float(e), i1)
        m2 = jnp.full((16,), _NEG, jnp.float32)
        for e in range(E):
            m2 = jnp.maximum(m2, jnp.where(i1 == float(e), _NEG, v[e]))
        i2 = jnp.zeros((16,), jnp.float32)
        for e in range(E - 1, -1, -1):
            i2 = jnp.where((v[e] == m2) & (i1 != float(e)), float(e), i2)
        t2 = jnp.exp(m2 - m1)
        ga = 1.0 / (1.0 + t2)
        gb = 1.0 - ga
        for e in range(E):
            ge = jnp.where(i1 == float(e), ga,
                           jnp.where(i2 == float(e), gb, 0.0))
            gbuf[e, pl.ds(t * 16, 16)] = ge
    pltpu.sync_copy(gbuf, gt_hbm.at[:, pl.ds(base, bw)])


def _stage3(hs_ref, gt_ref, w2_ref, b2_ref, sel_ref, out_ref):
    gt = gt_ref[...]
    cdims = (((0,), (0,)), ((), ()))
    g_exp = lax.dot_general(gt, sel_ref[...], cdims,
                            preferred_element_type=jnp.float32)
    hg = hs_ref[...] * g_exp.astype(jnp.bfloat16)
    out = jnp.dot(hg, w2_ref[...].astype(jnp.bfloat16),
                  preferred_element_type=jnp.float32)
    out_ref[...] = out + lax.dot_general(
        gt, b2_ref[...], cdims, preferred_element_type=jnp.float32)


def kernel(x, Wg, bg, W1, b1, W2, b2):
    B, D = x.shape
    E = Wg.shape[-1]
    H = W1.shape[-1]
    EH = E * H
    bg2 = bg.reshape(1, E)
    w1_all = jnp.transpose(W1, (1, 0, 2)).reshape(D, EH)
    b1_all = b1.reshape(1, EH)
    w2_all = W2.reshape(EH, D)
    sel = 0.5 * (jnp.arange(EH)[None, :] // H
                 == jnp.arange(E)[:, None]).astype(jnp.float32)

    grid = (B // _BB,)
    const = lambda i: (0, 0)
    hs, logitsT = pl.pallas_call(
        _stage1,
        grid=grid,
        in_specs=[
            pl.BlockSpec((_BB, D), lambda i: (i, 0)),
            pl.BlockSpec((D, E), const),
            pl.BlockSpec((1, E), const),
            pl.BlockSpec((D, EH), const),
            pl.BlockSpec((1, EH), const),
        ],
        out_specs=[
            pl.BlockSpec((_BB, EH), lambda i: (i, 0)),
            pl.BlockSpec((E, _BB), lambda i: (0, i)),
        ],
        out_shape=[
            jax.ShapeDtypeStruct((B, EH), jnp.bfloat16),
            jax.ShapeDtypeStruct((E, B), jnp.float32),
        ],
        compiler_params=pltpu.CompilerParams(
            dimension_semantics=("arbitrary",),
        ),
    )(x, Wg, bg2, w1_all, b1_all)

    mesh = plsc.VectorSubcoreMesh(core_axis_name="c", subcore_axis_name="s")
    gatT = functools.partial(
        pl.kernel,
        mesh=mesh,
        out_type=jax.ShapeDtypeStruct((E, B), jnp.float32),
        scratch_types=[
            pltpu.VMEM((E, B // 32), jnp.float32),
            pltpu.VMEM((E, B // 32), jnp.float32),
        ],
    )(_sc_gating)(logitsT)

    out = pl.pallas_call(
        _stage3,
        grid=grid,
        in_specs=[
            pl.BlockSpec((_BB, EH), lambda i: (i, 0)),
            pl.BlockSpec((E, _BB), lambda i: (0, i)),
            pl.BlockSpec((EH, D), const),
            pl.BlockSpec((E, D), const),
            pl.BlockSpec((E, EH), const),
        ],
        out_specs=pl.BlockSpec((_BB, D), lambda i: (i, 0)),
        out_shape=jax.ShapeDtypeStruct((B, D), jnp.float32),
        compiler_params=pltpu.CompilerParams(
            dimension_semantics=("arbitrary",),
        ),
    )(hs, gatT, w2_all, b2, sel)
    return out


# speedup vs baseline: 1.0516x; 1.0516x over previous
import functools

import jax
import jax.numpy as jnp
from jax import lax
from jax.experimental import pallas as pl
from jax.experimental.pallas import tpu as pltpu
from jax.experimental.pallas import tpu_sc as plsc

_KTOP = 2
_NEG = float(jnp.finfo(jnp.float32).min)
_BB = 2048


def _stage1(x_ref, wg_ref, bg_ref, w1_ref, b1_ref, hs_ref, lt_ref):
    x = x_ref[...]
    logits = jnp.dot(x, wg_ref[...], preferred_element_type=jnp.float32)
    logits = logits + bg_ref[...]
    lt_ref[...] = logits.T
    xb = x.astype(jnp.bfloat16)
    h = jnp.dot(xb, w1_ref[...].astype(jnp.bfloat16),
                preferred_element_type=jnp.float32)
    h = h + b1_ref[...]
    hs_ref[...] = (h * (1.0 + lax.erf(h * 0.7071067811865476))
                   ).astype(jnp.bfloat16)


def _sc_gating(lt_hbm, gt_hbm, lbuf, gbuf):
    E = lt_hbm.shape[0]
    B = lt_hbm.shape[1]
    nc = 2
    bw = B // 32
    wid = lax.axis_index("s") * nc + lax.axis_index("c")
    base = wid * bw
    pltpu.sync_copy(lt_hbm.at[:, pl.ds(base, bw)], lbuf)
    for t in range(bw // 16):
        v = [lbuf[e, pl.ds(t * 16, 16)] for e in range(E)]
        m1 = v[0]
        for e in range(1, E):
            m1 = jnp.maximum(m1, v[e])
        i1 = jnp.zeros((16,), jnp.float32)
        for e in range(E - 1, -1, -1):
            i1 = jnp.where(v[e] == m1, float(e), i1)
        m2 = jnp.full((16,), _NEG, jnp.float32)
        for e in range(E):
            m2 = jnp.maximum(m2, jnp.where(i1 == float(e), _NEG, v[e]))
        i2 = jnp.zeros((16,), jnp.float32)
        for e in range(E - 1, -1, -1):
            i2 = jnp.where((v[e] == m2) & (i1 != float(e)), float(e), i2)
        t2 = jnp.exp(m2 - m1)
        ga = 1.0 / (1.0 + t2)
        gb = 1.0 - ga
        for e in range(E):
            ge = jnp.where(i1 == float(e), ga,
                           jnp.where(i2 == float(e), gb, 0.0))
            gbuf[e, pl.ds(t * 16, 16)] = ge
    pltpu.sync_copy(gbuf, gt_hbm.at[:, pl.ds(base, bw)])


def _stage3(hs_ref, gt_ref, w2_ref, b2_ref, sel_ref, out_ref):
    gt = gt_ref[...]
    cdims = (((0,), (0,)), ((), ()))
    g_exp = lax.dot_general(gt, sel_ref[...], cdims,
                            preferred_element_type=jnp.float32)
    hg = hs_ref[...] * g_exp.astype(jnp.bfloat16)
    out = jnp.dot(hg, w2_ref[...].astype(jnp.bfloat16),
                  preferred_element_type=jnp.float32)
    out_ref[...] = out + lax.dot_general(
        gt, b2_ref[...], cdims, preferred_element_type=jnp.float32)


def kernel(x, Wg, bg, W1, b1, W2, b2):
    B, D = x.shape
    E = Wg.shape[-1]
    H = W1.shape[-1]
    EH = E * H
    bg2 = bg.reshape(1, E)
    w1_all = jnp.transpose(W1, (1, 0, 2)).reshape(D, EH)
    b1_all = b1.reshape(1, EH)
    w2_all = W2.reshape(EH, D)
    sel = 0.5 * (jnp.arange(EH)[None, :] // H
                 == jnp.arange(E)[:, None]).astype(jnp.float32)

    grid = (B // _BB,)
    const = lambda i: (0, 0)
    hs, logitsT = pl.pallas_call(
        _stage1,
        grid=grid,
        in_specs=[
            pl.BlockSpec((_BB, D), lambda i: (i, 0)),
            pl.BlockSpec((D, E), const),
            pl.BlockSpec((1, E), const),
            pl.BlockSpec((D, EH), const),
            pl.BlockSpec((1, EH), const),
        ],
        out_specs=[
            pl.BlockSpec((_BB, EH), lambda i: (i, 0)),
            pl.BlockSpec((E, _BB), lambda i: (0, i)),
        ],
        out_shape=[
            jax.ShapeDtypeStruct((B, EH), jnp.bfloat16),
            jax.ShapeDtypeStruct((E, B), jnp.float32),
        ],
        compiler_params=pltpu.CompilerParams(
            dimension_semantics=("arbitrary",),
        ),
    )(x, Wg, bg2, w1_all, b1_all)

    mesh = plsc.VectorSubcoreMesh(core_axis_name="c", subcore_axis_name="s")
    gatT = functools.partial(
        pl.kernel,
        mesh=mesh,
        out_type=jax.ShapeDtypeStruct((E, B), jnp.float32),
        scratch_types=[
            pltpu.VMEM((E, B // 32), jnp.float32),
            pltpu.VMEM((E, B // 32), jnp.float32),
        ],
    )(_sc_gating)(logitsT)

    out = pl.pallas_call(
        _stage3,
        grid=grid,
        in_specs=[
            pl.BlockSpec((_BB, EH), lambda i: (i, 0)),
            pl.BlockSpec((E, _BB), lambda i: (0, i)),
            pl.BlockSpec((EH, D), const),
            pl.BlockSpec((E, D), const),
            pl.BlockSpec((E, EH), const),
        ],
        out_specs=pl.BlockSpec((_BB, D), lambda i: (i, 0)),
        out_shape=jax.ShapeDtypeStruct((B, D), jnp.float32),
        compiler_params=pltpu.CompilerParams(
            dimension_semantics=("arbitrary",),
        ),
    )(hs, gatT, w2_all, b2, sel)
    return out
